# baseline (speedup 1.0000x reference)
import jax
import jax.numpy as jnp
from jax import lax
from jax.experimental import pallas as pl
from jax.experimental.pallas import tpu as pltpu

H = 32
W = 32
D = 256
HW = H * W


def _pos_kernel(col_ref, row_ref, out_ref, scratch):
    b = pl.program_id(0)

    @pl.when(b == 0)
    def _build():
        i0 = lax.broadcasted_iota(jnp.int32, (W, HW), 0)
        i1 = lax.broadcasted_iota(jnp.int32, (W, HW), 1)
        p_tile = (lax.bitwise_and(i1, W - 1) == i0).astype(jnp.float32)
        p_rep = (lax.shift_right_logical(i1, 5) == i0).astype(jnp.float32)
        dn = (((0,), (0,)), ((), ()))
        scratch[0:D, :] = lax.dot_general(
            col_ref[0:W, :], p_tile, dn, preferred_element_type=jnp.float32)
        scratch[D:2 * D, :] = lax.dot_general(
            row_ref[0:W, :], p_rep, dn, preferred_element_type=jnp.float32)

    out_ref[0] = scratch[...]


def kernel(x, row_embed, col_embed):
    batch = x.shape[0]
    out3 = pl.pallas_call(
        _pos_kernel,
        grid=(batch,),
        in_specs=[
            pl.BlockSpec((50, D), lambda b: (0, 0)),
            pl.BlockSpec((50, D), lambda b: (0, 0)),
        ],
        out_specs=pl.BlockSpec((1, 2 * D, HW), lambda b: (b, 0, 0)),
        out_shape=jax.ShapeDtypeStruct((batch, 2 * D, HW), jnp.float32),
        scratch_shapes=[pltpu.VMEM((2 * D, HW), jnp.float32)],
    )(col_embed, row_embed)
    return out3.reshape(batch, 2 * D, H, W)


# speedup vs baseline: 1.0371x; 1.0371x over previous
import jax
import jax.numpy as jnp
from jax import lax
from jax.experimental import pallas as pl
from jax.experimental.pallas import tpu as pltpu

H = 32
W = 32
D = 256
HW = H * W


def _pos_kernel(col_ref, row_ref, out_ref, scratch, sem):
    i0 = lax.broadcasted_iota(jnp.int32, (W, HW), 0)
    i1 = lax.broadcasted_iota(jnp.int32, (W, HW), 1)
    p_tile = (lax.bitwise_and(i1, W - 1) == i0).astype(jnp.float32)
    p_rep = (lax.shift_right_logical(i1, 5) == i0).astype(jnp.float32)
    dn = (((0,), (0,)), ((), ()))
    scratch[0:D, :] = lax.dot_general(
        col_ref[0:W, :], p_tile, dn, preferred_element_type=jnp.float32)
    scratch[D:2 * D, :] = lax.dot_general(
        row_ref[0:W, :], p_rep, dn, preferred_element_type=jnp.float32)

    batch = out_ref.shape[0]
    for b in range(batch):
        pltpu.make_async_copy(scratch, out_ref.at[b], sem).start()
    for b in range(batch):
        pltpu.make_async_copy(scratch, out_ref.at[b], sem).wait()


def kernel(x, row_embed, col_embed):
    batch = x.shape[0]
    out3 = pl.pallas_call(
        _pos_kernel,
        in_specs=[
            pl.BlockSpec(memory_space=pltpu.VMEM),
            pl.BlockSpec(memory_space=pltpu.VMEM),
        ],
        out_specs=pl.BlockSpec(memory_space=pl.ANY),
        out_shape=jax.ShapeDtypeStruct((batch, 2 * D, HW), jnp.float32),
        scratch_shapes=[
            pltpu.VMEM((2 * D, HW), jnp.float32),
            pltpu.SemaphoreType.DMA,
        ],
    )(col_embed, row_embed)
    return out3.reshape(batch, 2 * D, H, W)
